# Initial kernel scaffold; baseline (speedup 1.0000x reference)
import jax
import jax.numpy as jnp
from jax.experimental import pallas as pl


def kernel(element_idxs, neighbor_idxs, distances, base_charges):
    raise NotImplementedError("write your pallas kernel here")


# speedup vs baseline: 374.1969x; 374.1969x over previous
import functools

import jax
import jax.numpy as jnp
from jax import lax
from jax.experimental import pallas as pl
from jax.experimental.pallas import tpu as pltpu
from jax.experimental.pallas import tpu_sc as plsc

NC = 2
NS = 16
L = 16
NW = NC * NS


def _make_kernel(n_atoms, n_pairs, chunk):
  n_tiles = NW
  pairs_per_tile = n_pairs // n_tiles
  n_chunks = pairs_per_tile // chunk
  atom_iters = n_atoms // L
  inv_n = 1.0 / float(n_atoms)

  mesh = plsc.VectorSubcoreMesh(
      core_axis_name="c", subcore_axis_name="s",
      num_cores=NC, num_subcores=NS)

  @functools.partial(
      pl.kernel,
      out_type=jax.ShapeDtypeStruct((n_pairs,), jnp.float32),
      mesh=mesh,
      compiler_params=pltpu.CompilerParams(needs_layout_passes=False),
      scratch_types=[
          pltpu.VMEM((n_atoms,), jnp.int32),
          pltpu.VMEM((L,), jnp.float32),
          pltpu.VMEM((chunk,), jnp.int32),
          pltpu.VMEM((chunk,), jnp.int32),
          pltpu.VMEM((chunk,), jnp.float32),
      ],
  )
  def k(elem_hbm, nbr_i_hbm, nbr_j_hbm, base_hbm, out_hbm,
        table_v, base_v, idx_i_v, idx_j_v, out_v):
    wid = lax.axis_index("s") * NC + lax.axis_index("c")

    pltpu.sync_copy(base_hbm, base_v)
    pltpu.sync_copy(elem_hbm, table_v)

    def atom_body(i, acc):
      sl = pl.ds(i * L, L)
      e = table_v[sl]
      c = plsc.load_gather(base_v, [e])
      table_v[sl] = plsc.bitcast(c, jnp.int32)
      return acc + c

    acc = lax.fori_loop(0, atom_iters, atom_body,
                        jnp.zeros((L,), jnp.float32))
    m = jnp.sum(acc) * inv_n
    m_vec = jnp.full((L,), m, jnp.float32)

    pair_base = wid * pairs_per_tile

    def chunk_body(g, _):
      off = pair_base + g * chunk
      pltpu.sync_copy(nbr_i_hbm.at[pl.ds(off, chunk)], idx_i_v)
      pltpu.sync_copy(nbr_j_hbm.at[pl.ds(off, chunk)], idx_j_v)

      def pair_body(t, _):
        sl = pl.ds(t * L, L)
        qi = plsc.bitcast(plsc.load_gather(table_v, [idx_i_v[sl]]),
                          jnp.float32)
        qj = plsc.bitcast(plsc.load_gather(table_v, [idx_j_v[sl]]),
                          jnp.float32)
        out_v[sl] = (qi - m_vec) * (qj - m_vec)
        return 0

      lax.fori_loop(0, chunk // L, pair_body, 0)
      pltpu.sync_copy(out_v, out_hbm.at[pl.ds(off, chunk)])
      return 0

    lax.fori_loop(0, n_chunks, chunk_body, 0)

  return k


@jax.jit
def kernel(element_idxs, neighbor_idxs, distances, base_charges):
  del distances
  b, n_atoms = element_idxs.shape
  n_pairs = neighbor_idxs.shape[1]
  elem = element_idxs.reshape(n_atoms).astype(jnp.int32)
  nbr_i = neighbor_idxs[0].astype(jnp.int32)
  nbr_j = neighbor_idxs[1].astype(jnp.int32)
  base = jnp.zeros((L,), jnp.float32).at[:base_charges.shape[0]].set(
      base_charges.astype(jnp.float32))
  k = _make_kernel(n_atoms, n_pairs, chunk=4000)
  out = k(elem, nbr_i, nbr_j, base)
  return out.reshape(b, n_pairs)


# speedup vs baseline: 594.8568x; 1.5897x over previous
import functools

import jax
import jax.numpy as jnp
from jax import lax
from jax.experimental import pallas as pl
from jax.experimental.pallas import tpu as pltpu
from jax.experimental.pallas import tpu_sc as plsc

NC = 2
NS = 16
L = 16
NW = NC * NS
U = 5


def _make_kernel(n_atoms, n_pairs, chunk, n_segs):
  pairs_per_tile = n_pairs // NW
  n_chunks = pairs_per_tile // chunk
  seg = n_atoms // n_segs
  assert n_chunks % 2 == 0 and chunk % (L * U) == 0 and seg % (L * U) == 0

  mesh = plsc.VectorSubcoreMesh(
      core_axis_name="c", subcore_axis_name="s",
      num_cores=NC, num_subcores=NS)

  @functools.partial(
      pl.kernel,
      out_type=jax.ShapeDtypeStruct((n_pairs,), jnp.float32),
      mesh=mesh,
      compiler_params=pltpu.CompilerParams(needs_layout_passes=False),
      scratch_types=[
          pltpu.VMEM((n_atoms,), jnp.int32),
          pltpu.VMEM((L,), jnp.float32),
          pltpu.VMEM((chunk,), jnp.int32),
          pltpu.VMEM((chunk,), jnp.int32),
          pltpu.VMEM((chunk,), jnp.int32),
          pltpu.VMEM((chunk,), jnp.int32),
          pltpu.VMEM((chunk,), jnp.float32),
          pltpu.VMEM((chunk,), jnp.float32),
          pltpu.SemaphoreType.DMA((2,)),
          pltpu.SemaphoreType.DMA((2,)),
          pltpu.SemaphoreType.DMA((2,)),
      ],
  )
  def k(elem_hbm, nbr_i_hbm, nbr_j_hbm, base_hbm, out_hbm,
        table_v, base_v, idx_i0, idx_i1, idx_j0, idx_j1, out0, out1,
        sem_in, sem_out, sem_t):
    wid = lax.axis_index("s") * NC + lax.axis_index("c")
    idx_i_b = (idx_i0, idx_i1)
    idx_j_b = (idx_j0, idx_j1)
    out_b = (out0, out1)

    pltpu.sync_copy(base_hbm, base_v)
    pltpu.async_copy(elem_hbm.at[pl.ds(0, seg)],
                     table_v.at[pl.ds(0, seg)], sem_t.at[0])
    acc = jnp.zeros((L,), jnp.float32)
    for s in range(n_segs):
      pltpu.make_async_copy(elem_hbm.at[pl.ds(s * seg, seg)],
                            table_v.at[pl.ds(s * seg, seg)],
                            sem_t.at[s % 2]).wait()
      if s + 1 < n_segs:
        pltpu.async_copy(elem_hbm.at[pl.ds((s + 1) * seg, seg)],
                         table_v.at[pl.ds((s + 1) * seg, seg)],
                         sem_t.at[(s + 1) % 2])

      def atom_body(i, acc, s=s):
        base_off = s * seg + i * (L * U)
        for u in range(U):
          sl = pl.ds(base_off + u * L, L)
          c = plsc.load_gather(base_v, [table_v[sl]])
          table_v[sl] = plsc.bitcast(c, jnp.int32)
          acc = acc + c
        return acc

      acc = lax.fori_loop(0, seg // (L * U), atom_body, acc)

    m = jnp.sum(acc) * (1.0 / float(n_atoms))
    m_vec = jnp.full((L,), m, jnp.float32)

    pair_base = wid * pairs_per_tile

    def start_in(g, b):
      off = pair_base + g * chunk
      pltpu.async_copy(nbr_i_hbm.at[pl.ds(off, chunk)], idx_i_b[b],
                       sem_in.at[b])
      pltpu.async_copy(nbr_j_hbm.at[pl.ds(off, chunk)], idx_j_b[b],
                       sem_in.at[b])

    def wait_in(g, b):
      off = pair_base + g * chunk
      pltpu.make_async_copy(nbr_i_hbm.at[pl.ds(off, chunk)], idx_i_b[b],
                            sem_in.at[b]).wait()
      pltpu.make_async_copy(nbr_j_hbm.at[pl.ds(off, chunk)], idx_j_b[b],
                            sem_in.at[b]).wait()

    def start_out(g, b):
      off = pair_base + g * chunk
      pltpu.async_copy(out_b[b], out_hbm.at[pl.ds(off, chunk)],
                       sem_out.at[b])

    def wait_out(g, b):
      off = pair_base + g * chunk
      pltpu.make_async_copy(out_b[b], out_hbm.at[pl.ds(off, chunk)],
                            sem_out.at[b]).wait()

    start_in(0, 0)
    start_in(1, 1)

    def chunk_pair_body(gg, _):
      for b in range(2):
        g = gg * 2 + b
        wait_in(g, b)

        @pl.when(gg > 0)
        def _():
          wait_out(g - 2, b)

        ib = idx_i_b[b]
        jb = idx_j_b[b]
        ob = out_b[b]

        def pair_body(t, _, ib=ib, jb=jb, ob=ob):
          base_t = t * (L * U)
          for u in range(U):
            sl = pl.ds(base_t + u * L, L)
            qi = plsc.bitcast(plsc.load_gather(table_v, [ib[sl]]),
                              jnp.float32)
            qj = plsc.bitcast(plsc.load_gather(table_v, [jb[sl]]),
                              jnp.float32)
            ob[sl] = (qi - m_vec) * (qj - m_vec)
          return 0

        lax.fori_loop(0, chunk // (L * U), pair_body, 0)
        start_out(g, b)

        @pl.when(g + 2 < n_chunks)
        def _():
          start_in(g + 2, b)
      return 0

    lax.fori_loop(0, n_chunks // 2, chunk_pair_body, 0)
    wait_out(n_chunks - 2, 0)
    wait_out(n_chunks - 1, 1)

  return k


@jax.jit
def kernel(element_idxs, neighbor_idxs, distances, base_charges):
  del distances
  b, n_atoms = element_idxs.shape
  n_pairs = neighbor_idxs.shape[1]
  elem = element_idxs.reshape(n_atoms).astype(jnp.int32)
  nbr_i = neighbor_idxs[0].astype(jnp.int32)
  nbr_j = neighbor_idxs[1].astype(jnp.int32)
  base = jnp.zeros((L,), jnp.float32).at[:base_charges.shape[0]].set(
      base_charges.astype(jnp.float32))
  k = _make_kernel(n_atoms, n_pairs, chunk=4000, n_segs=10)
  out = k(elem, nbr_i, nbr_j, base)
  return out.reshape(b, n_pairs)


# speedup vs baseline: 876.2469x; 1.4730x over previous
import functools

import jax
import jax.numpy as jnp
from jax import lax
from jax.experimental import pallas as pl
from jax.experimental.pallas import tpu as pltpu
from jax.experimental.pallas import tpu_sc as plsc

NC = 2
NS = 16
L = 16
NW = NC * NS
U = 5


def _make_kernel(n_atoms, n_pairs, chunk, n_segs):
  pairs_per_tile = n_pairs // NW
  n_chunks = pairs_per_tile // chunk
  seg = n_atoms // n_segs
  assert n_chunks % 2 == 0 and chunk % (L * U) == 0 and seg % (L * U) == 0

  mesh = plsc.VectorSubcoreMesh(
      core_axis_name="c", subcore_axis_name="s",
      num_cores=NC, num_subcores=NS)

  @functools.partial(
      pl.kernel,
      out_type=jax.ShapeDtypeStruct((n_pairs,), jnp.float32),
      mesh=mesh,
      compiler_params=pltpu.CompilerParams(needs_layout_passes=False),
      scratch_types=[
          pltpu.VMEM((n_atoms,), jnp.int32),
          pltpu.VMEM((L,), jnp.float32),
          pltpu.VMEM((chunk,), jnp.int32),
          pltpu.VMEM((chunk,), jnp.int32),
          pltpu.VMEM((chunk,), jnp.int32),
          pltpu.VMEM((chunk,), jnp.int32),
          pltpu.VMEM((chunk,), jnp.float32),
          pltpu.VMEM((chunk,), jnp.float32),
          pltpu.SemaphoreType.DMA((2,)),
          pltpu.SemaphoreType.DMA((2,)),
          pltpu.SemaphoreType.DMA((2,)),
      ],
  )
  def k(elem_hbm, nbr_i_hbm, nbr_j_hbm, base_hbm, out_hbm,
        table_v, base_v, idx_i0, idx_i1, idx_j0, idx_j1, out0, out1,
        sem_in, sem_out, sem_t):
    wid = lax.axis_index("s") * NC + lax.axis_index("c")
    idx_i_b = (idx_i0, idx_i1)
    idx_j_b = (idx_j0, idx_j1)
    out_b = (out0, out1)

    pltpu.sync_copy(base_hbm, base_v)
    pltpu.async_copy(elem_hbm.at[pl.ds(0, seg)],
                     table_v.at[pl.ds(0, seg)], sem_t.at[0])
    acc = jnp.zeros((L,), jnp.float32)
    for s in range(n_segs):
      pltpu.make_async_copy(elem_hbm.at[pl.ds(s * seg, seg)],
                            table_v.at[pl.ds(s * seg, seg)],
                            sem_t.at[s % 2]).wait()
      if s + 1 < n_segs:
        pltpu.async_copy(elem_hbm.at[pl.ds((s + 1) * seg, seg)],
                         table_v.at[pl.ds((s + 1) * seg, seg)],
                         sem_t.at[(s + 1) % 2])

      def atom_body(i, acc, s=s):
        base_off = s * seg + i * (L * U)
        sls = [pl.ds(base_off + u * L, L) for u in range(U)]
        es = [table_v[sl] for sl in sls]
        cs = [plsc.load_gather(base_v, [e]) for e in es]
        for sl, c in zip(sls, cs):
          table_v[sl] = plsc.bitcast(c, jnp.int32)
        for c in cs:
          acc = acc + c
        return acc

      acc = lax.fori_loop(0, seg // (L * U), atom_body, acc)

    m = jnp.sum(acc) * (1.0 / float(n_atoms))
    m_vec = jnp.full((L,), m, jnp.float32)

    pair_base = wid * pairs_per_tile

    def start_in(g, b):
      off = pair_base + g * chunk
      pltpu.async_copy(nbr_i_hbm.at[pl.ds(off, chunk)], idx_i_b[b],
                       sem_in.at[b])
      pltpu.async_copy(nbr_j_hbm.at[pl.ds(off, chunk)], idx_j_b[b],
                       sem_in.at[b])

    def wait_in(g, b):
      off = pair_base + g * chunk
      pltpu.make_async_copy(nbr_i_hbm.at[pl.ds(off, chunk)], idx_i_b[b],
                            sem_in.at[b]).wait()
      pltpu.make_async_copy(nbr_j_hbm.at[pl.ds(off, chunk)], idx_j_b[b],
                            sem_in.at[b]).wait()

    def start_out(g, b):
      off = pair_base + g * chunk
      pltpu.async_copy(out_b[b], out_hbm.at[pl.ds(off, chunk)],
                       sem_out.at[b])

    def wait_out(g, b):
      off = pair_base + g * chunk
      pltpu.make_async_copy(out_b[b], out_hbm.at[pl.ds(off, chunk)],
                            sem_out.at[b]).wait()

    start_in(0, 0)
    start_in(1, 1)

    def chunk_pair_body(gg, _):
      for b in range(2):
        g = gg * 2 + b
        wait_in(g, b)

        @pl.when(gg > 0)
        def _():
          wait_out(g - 2, b)

        ib = idx_i_b[b]
        jb = idx_j_b[b]
        ob = out_b[b]

        def pair_body(t, _, ib=ib, jb=jb, ob=ob):
          base_t = t * (L * U)
          sls = [pl.ds(base_t + u * L, L) for u in range(U)]
          iis = [ib[sl] for sl in sls]
          jjs = [jb[sl] for sl in sls]
          qis = [plsc.bitcast(plsc.load_gather(table_v, [x]), jnp.float32)
                 for x in iis]
          qjs = [plsc.bitcast(plsc.load_gather(table_v, [x]), jnp.float32)
                 for x in jjs]
          ps = [(qi - m_vec) * (qj - m_vec) for qi, qj in zip(qis, qjs)]
          for sl, p in zip(sls, ps):
            ob[sl] = p
          return 0

        lax.fori_loop(0, chunk // (L * U), pair_body, 0)
        start_out(g, b)

        @pl.when(g + 2 < n_chunks)
        def _():
          start_in(g + 2, b)
      return 0

    lax.fori_loop(0, n_chunks // 2, chunk_pair_body, 0)
    wait_out(n_chunks - 2, 0)
    wait_out(n_chunks - 1, 1)

  return k


@jax.jit
def kernel(element_idxs, neighbor_idxs, distances, base_charges):
  del distances
  b, n_atoms = element_idxs.shape
  n_pairs = neighbor_idxs.shape[1]
  elem = element_idxs.reshape(n_atoms).astype(jnp.int32)
  nbr_i = neighbor_idxs[0].astype(jnp.int32)
  nbr_j = neighbor_idxs[1].astype(jnp.int32)
  base = jnp.zeros((L,), jnp.float32).at[:base_charges.shape[0]].set(
      base_charges.astype(jnp.float32))
  k = _make_kernel(n_atoms, n_pairs, chunk=4000, n_segs=10)
  out = k(elem, nbr_i, nbr_j, base)
  return out.reshape(b, n_pairs)


# speedup vs baseline: 1001.3820x; 1.1428x over previous
import functools

import jax
import jax.numpy as jnp
from jax import lax
from jax.experimental import pallas as pl
from jax.experimental.pallas import tpu as pltpu
from jax.experimental.pallas import tpu_sc as plsc

NC = 2
NS = 16
L = 16
NW = NC * NS
U = 5


def _make_kernel(n_atoms, n_pairs, chunk, n_segs):
  pairs_per_tile = n_pairs // NW
  n_chunks = pairs_per_tile // chunk
  seg = n_atoms // n_segs
  assert n_chunks % 2 == 0 and chunk % (L * U) == 0 and seg % (L * U) == 0

  mesh = plsc.VectorSubcoreMesh(
      core_axis_name="c", subcore_axis_name="s",
      num_cores=NC, num_subcores=NS)

  @functools.partial(
      pl.kernel,
      out_type=jax.ShapeDtypeStruct((n_pairs,), jnp.float32),
      mesh=mesh,
      compiler_params=pltpu.CompilerParams(needs_layout_passes=False),
      scratch_types=[
          pltpu.VMEM((n_atoms,), jnp.int32),
          pltpu.VMEM((L,), jnp.float32),
          pltpu.VMEM((chunk,), jnp.int32),
          pltpu.VMEM((chunk,), jnp.int32),
          pltpu.VMEM((chunk,), jnp.int32),
          pltpu.VMEM((chunk,), jnp.int32),
          pltpu.VMEM((chunk,), jnp.float32),
          pltpu.VMEM((chunk,), jnp.float32),
          pltpu.SemaphoreType.DMA((2,)),
          pltpu.SemaphoreType.DMA((2,)),
          pltpu.SemaphoreType.DMA((2,)),
      ],
  )
  def k(elem_hbm, nbr_hbm, base_hbm, out_hbm,
        table_v, base_v, idx_i0, idx_i1, idx_j0, idx_j1, out0, out1,
        sem_in, sem_out, sem_t):
    wid = lax.axis_index("s") * NC + lax.axis_index("c")
    idx_i_b = (idx_i0, idx_i1)
    idx_j_b = (idx_j0, idx_j1)
    out_b = (out0, out1)

    pltpu.sync_copy(base_hbm, base_v)
    pltpu.async_copy(elem_hbm.at[pl.ds(0, seg)],
                     table_v.at[pl.ds(0, seg)], sem_t.at[0])
    acc = jnp.zeros((L,), jnp.float32)
    for s in range(n_segs):
      pltpu.make_async_copy(elem_hbm.at[pl.ds(s * seg, seg)],
                            table_v.at[pl.ds(s * seg, seg)],
                            sem_t.at[s % 2]).wait()
      if s + 1 < n_segs:
        pltpu.async_copy(elem_hbm.at[pl.ds((s + 1) * seg, seg)],
                         table_v.at[pl.ds((s + 1) * seg, seg)],
                         sem_t.at[(s + 1) % 2])

      def atom_body(i, acc, s=s):
        base_off = s * seg + i * (L * U)
        sls = [pl.ds(base_off + u * L, L) for u in range(U)]
        es = [table_v[sl] for sl in sls]
        cs = [plsc.load_gather(base_v, [e]) for e in es]
        for sl, c in zip(sls, cs):
          table_v[sl] = plsc.bitcast(c, jnp.int32)
        for c in cs:
          acc = acc + c
        return acc

      acc = lax.fori_loop(0, seg // (L * U), atom_body, acc)

    m = jnp.sum(acc) * (1.0 / float(n_atoms))
    m_vec = jnp.full((L,), m, jnp.float32)

    pair_base = wid * pairs_per_tile

    def start_in(g, b):
      off = pair_base + g * chunk
      pltpu.async_copy(nbr_hbm.at[pl.ds(off, chunk)], idx_i_b[b],
                       sem_in.at[b])
      pltpu.async_copy(nbr_hbm.at[pl.ds(n_pairs + off, chunk)], idx_j_b[b],
                       sem_in.at[b])

    def wait_in(g, b):
      off = pair_base + g * chunk
      pltpu.make_async_copy(nbr_hbm.at[pl.ds(off, chunk)], idx_i_b[b],
                            sem_in.at[b]).wait()
      pltpu.make_async_copy(nbr_hbm.at[pl.ds(n_pairs + off, chunk)], idx_j_b[b],
                            sem_in.at[b]).wait()

    def start_out(g, b):
      off = pair_base + g * chunk
      pltpu.async_copy(out_b[b], out_hbm.at[pl.ds(off, chunk)],
                       sem_out.at[b])

    def wait_out(g, b):
      off = pair_base + g * chunk
      pltpu.make_async_copy(out_b[b], out_hbm.at[pl.ds(off, chunk)],
                            sem_out.at[b]).wait()

    start_in(0, 0)
    start_in(1, 1)

    def chunk_pair_body(gg, _):
      for b in range(2):
        g = gg * 2 + b
        wait_in(g, b)

        @pl.when(gg > 0)
        def _():
          wait_out(g - 2, b)

        ib = idx_i_b[b]
        jb = idx_j_b[b]
        ob = out_b[b]

        def pair_body(t, _, ib=ib, jb=jb, ob=ob):
          base_t = t * (L * U)
          sls = [pl.ds(base_t + u * L, L) for u in range(U)]
          iis = [ib[sl] for sl in sls]
          jjs = [jb[sl] for sl in sls]
          qis = [plsc.bitcast(plsc.load_gather(table_v, [x]), jnp.float32)
                 for x in iis]
          qjs = [plsc.bitcast(plsc.load_gather(table_v, [x]), jnp.float32)
                 for x in jjs]
          ps = [(qi - m_vec) * (qj - m_vec) for qi, qj in zip(qis, qjs)]
          for sl, p in zip(sls, ps):
            ob[sl] = p
          return 0

        lax.fori_loop(0, chunk // (L * U), pair_body, 0)
        start_out(g, b)

        @pl.when(g + 2 < n_chunks)
        def _():
          start_in(g + 2, b)
      return 0

    lax.fori_loop(0, n_chunks // 2, chunk_pair_body, 0)
    wait_out(n_chunks - 2, 0)
    wait_out(n_chunks - 1, 1)

  return k


@jax.jit
def kernel(element_idxs, neighbor_idxs, distances, base_charges):
  del distances
  b, n_atoms = element_idxs.shape
  n_pairs = neighbor_idxs.shape[1]
  elem = element_idxs.reshape(n_atoms).astype(jnp.int32)
  nbr = neighbor_idxs.reshape(2 * n_pairs).astype(jnp.int32)
  base = jnp.zeros((L,), jnp.float32).at[:base_charges.shape[0]].set(
      base_charges.astype(jnp.float32))
  k = _make_kernel(n_atoms, n_pairs, chunk=4000, n_segs=10)
  out = k(elem, nbr, base)
  return out.reshape(b, n_pairs)


# speedup vs baseline: 1299.8194x; 1.2980x over previous
import functools

import jax
import jax.numpy as jnp
from jax import lax
from jax.experimental import pallas as pl
from jax.experimental.pallas import tpu as pltpu
from jax.experimental.pallas import tpu_sc as plsc

NC = 2
NS = 16
L = 16
NW = NC * NS
U = 5


def _make_kernel(n_atoms, n_pairs, chunk, n_segs):
  n_chunks_total = n_pairs // chunk
  k_steps = -(-n_chunks_total // NW)
  k_steps += k_steps % 2
  seg = n_atoms // n_segs
  assert n_pairs % chunk == 0 and chunk % (L * U) == 0 and chunk % 128 == 0
  assert seg % (L * U) == 0

  mesh = plsc.VectorSubcoreMesh(
      core_axis_name="c", subcore_axis_name="s",
      num_cores=NC, num_subcores=NS)

  @functools.partial(
      pl.kernel,
      out_type=jax.ShapeDtypeStruct((n_pairs,), jnp.float32),
      mesh=mesh,
      compiler_params=pltpu.CompilerParams(needs_layout_passes=False),
      scratch_types=[
          pltpu.VMEM((n_atoms,), jnp.int32),
          pltpu.VMEM((L,), jnp.float32),
          pltpu.VMEM((chunk,), jnp.int32),
          pltpu.VMEM((chunk,), jnp.int32),
          pltpu.VMEM((chunk,), jnp.int32),
          pltpu.VMEM((chunk,), jnp.int32),
          pltpu.VMEM((chunk,), jnp.float32),
          pltpu.VMEM((chunk,), jnp.float32),
          pltpu.SemaphoreType.DMA((2,)),
          pltpu.SemaphoreType.DMA((2,)),
          pltpu.SemaphoreType.DMA((2,)),
      ],
  )
  def k(elem_hbm, nbr_hbm, base_hbm, out_hbm,
        table_v, base_v, idx_i0, idx_i1, idx_j0, idx_j1, out0, out1,
        sem_in, sem_out, sem_t):
    wid = lax.axis_index("s") * NC + lax.axis_index("c")
    idx_i_b = (idx_i0, idx_i1)
    idx_j_b = (idx_j0, idx_j1)
    out_b = (out0, out1)

    pltpu.sync_copy(base_hbm, base_v)
    pltpu.async_copy(elem_hbm.at[pl.ds(0, seg)],
                     table_v.at[pl.ds(0, seg)], sem_t.at[0])
    acc = jnp.zeros((L,), jnp.float32)
    for s in range(n_segs):
      pltpu.make_async_copy(elem_hbm.at[pl.ds(s * seg, seg)],
                            table_v.at[pl.ds(s * seg, seg)],
                            sem_t.at[s % 2]).wait()
      if s + 1 < n_segs:
        pltpu.async_copy(elem_hbm.at[pl.ds((s + 1) * seg, seg)],
                         table_v.at[pl.ds((s + 1) * seg, seg)],
                         sem_t.at[(s + 1) % 2])

      def atom_body(i, acc, s=s):
        base_off = s * seg + i * (L * U)
        sls = [pl.ds(base_off + u * L, L) for u in range(U)]
        es = [table_v[sl] for sl in sls]
        cs = [plsc.load_gather(base_v, [e]) for e in es]
        for sl, c in zip(sls, cs):
          table_v[sl] = plsc.bitcast(c, jnp.int32)
        for c in cs:
          acc = acc + c
        return acc

      acc = lax.fori_loop(0, seg // (L * U), atom_body, acc)

    m = jnp.sum(acc) * (1.0 / float(n_atoms))
    m_vec = jnp.full((L,), m, jnp.float32)

    def chunk_off(k):
      g = wid + NW * k
      return pl.multiple_of(g * chunk, chunk)

    def valid(k):
      return wid + NW * k < n_chunks_total

    def start_in(k, b):
      off = chunk_off(k)
      pltpu.async_copy(nbr_hbm.at[0, pl.ds(off, chunk)], idx_i_b[b],
                       sem_in.at[b])
      pltpu.async_copy(nbr_hbm.at[1, pl.ds(off, chunk)], idx_j_b[b],
                       sem_in.at[b])

    def wait_in(k, b):
      off = chunk_off(k)
      pltpu.make_async_copy(nbr_hbm.at[0, pl.ds(off, chunk)], idx_i_b[b],
                            sem_in.at[b]).wait()
      pltpu.make_async_copy(nbr_hbm.at[1, pl.ds(off, chunk)], idx_j_b[b],
                            sem_in.at[b]).wait()

    def start_out(k, b):
      off = chunk_off(k)
      pltpu.async_copy(out_b[b], out_hbm.at[pl.ds(off, chunk)],
                       sem_out.at[b])

    def wait_out(k, b):
      off = chunk_off(k)
      pltpu.make_async_copy(out_b[b], out_hbm.at[pl.ds(off, chunk)],
                            sem_out.at[b]).wait()

    start_in(0, 0)
    start_in(1, 1)

    def chunk_pair_body(kk, _):
      for b in range(2):
        k = kk * 2 + b

        @pl.when(valid(k))
        def _(k=k, b=b):
          wait_in(k, b)

          @pl.when(kk > 0)
          def _():
            wait_out(k - 2, b)

          ib = idx_i_b[b]
          jb = idx_j_b[b]
          ob = out_b[b]

          def pair_body(t, _, ib=ib, jb=jb, ob=ob):
            base_t = t * (L * U)
            sls = [pl.ds(base_t + u * L, L) for u in range(U)]
            iis = [ib[sl] for sl in sls]
            jjs = [jb[sl] for sl in sls]
            qis = [plsc.bitcast(plsc.load_gather(table_v, [x]), jnp.float32)
                   for x in iis]
            qjs = [plsc.bitcast(plsc.load_gather(table_v, [x]), jnp.float32)
                   for x in jjs]
            ps = [(qi - m_vec) * (qj - m_vec) for qi, qj in zip(qis, qjs)]
            for sl, p in zip(sls, ps):
              ob[sl] = p
            return 0

          lax.fori_loop(0, chunk // (L * U), pair_body, 0)
          start_out(k, b)

          @pl.when(valid(k + 2))
          def _():
            start_in(k + 2, b)
      return 0

    lax.fori_loop(0, k_steps // 2, chunk_pair_body, 0)

    for k in range(max(0, k_steps - 4), k_steps):
      @pl.when(valid(k) & jnp.logical_not(valid(k + 2)))
      def _(k=k):
        wait_out(k, k % 2)

  return k


@jax.jit
def kernel(element_idxs, neighbor_idxs, distances, base_charges):
  del distances
  b, n_atoms = element_idxs.shape
  n_pairs = neighbor_idxs.shape[1]
  elem = element_idxs.reshape(n_atoms).astype(jnp.int32)
  nbr = neighbor_idxs.astype(jnp.int32)
  base = jnp.zeros((L,), jnp.float32).at[:base_charges.shape[0]].set(
      base_charges.astype(jnp.float32))
  k = _make_kernel(n_atoms, n_pairs, chunk=2560, n_segs=10)
  out = k(elem, nbr, base)
  return out.reshape(b, n_pairs)
